# baseline (speedup 1.0000x reference)
import functools

import jax
import jax.numpy as jnp
from jax import lax
from jax.experimental import pallas as pl
from jax.experimental.pallas import tpu as pltpu
from jax.experimental.pallas import tpu_sc as plsc

_NUM_CLASSES = 100000
_EMBED_DIM = 32
_MULT = 4
_BATCH = 16384

_NW = 32
_TOTAL = _BATCH * _MULT
_B_PER_W = _TOTAL // _NW
_CHUNK = 128
_N_CHUNKS = _B_PER_W // _CHUNK
_LANES = 16


def _sc_embedding_lookup(feat_grouped, weight):
    mesh = plsc.VectorSubcoreMesh(core_axis_name="c", subcore_axis_name="s")

    @functools.partial(
        pl.kernel,
        mesh=mesh,
        out_type=jax.ShapeDtypeStruct(
            (_NW, _N_CHUNKS, _CHUNK, _EMBED_DIM), jnp.float32),
        scratch_types=[
            pltpu.VMEM((_N_CHUNKS, _CHUNK), jnp.int32),
            pltpu.VMEM((_N_CHUNKS, _CHUNK, _EMBED_DIM), jnp.float32),
            pltpu.SemaphoreType.DMA,
        ],
        compiler_params=pltpu.CompilerParams(use_tc_tiling_on_sc=False),
    )
    def body(feat_hbm, table_hbm, out_hbm, idx_v, rows_v, sem):
        wid = lax.axis_index("s") * 2 + lax.axis_index("c")
        pltpu.sync_copy(feat_hbm.at[wid], idx_v)

        off = (lax.iota(jnp.int32, 16) % _MULT) * _NUM_CLASSES

        copies = []
        for j in range(_N_CHUNKS):
            def add_off(s, carry, j=j):
                sl = pl.ds(s * _LANES, _LANES)
                idx_v[j, sl] = idx_v[j, sl] + off
                return carry
            lax.fori_loop(0, _CHUNK // _LANES, add_off, 0, unroll=True)
            copies.append(
                pltpu.async_copy(table_hbm.at[idx_v.at[j]], rows_v.at[j], sem))
        for c in copies:
            c.wait()
        pltpu.sync_copy(rows_v, out_hbm.at[wid])

    return body(feat_grouped, weight)


def kernel(feat, weight):
    feat_grouped = feat.reshape(_NW, _N_CHUNKS, _CHUNK)
    out = _sc_embedding_lookup(feat_grouped, weight)
    return out.reshape(_BATCH, _MULT, _EMBED_DIM)
